# baseline (speedup 1.0000x reference)
import functools

import jax
import jax.numpy as jnp
from jax import lax
from jax.experimental import pallas as pl
from jax.experimental.pallas import tpu as pltpu
from jax.experimental.pallas import tpu_sc as plsc

_B = 4096
_F = 100
_D = 128
_T = _F + 1
_TP = 112
_NC = 2
_NS = 16
_NW = _NC * _NS
_BPW = _B // _NW
_PAIRS = _BPW // 2

_mesh = plsc.VectorSubcoreMesh(core_axis_name="c", subcore_axis_name="s")


@functools.partial(
    pl.kernel,
    out_type=jax.ShapeDtypeStruct((_B, _T, _D), jnp.float32),
    mesh=_mesh,
    scratch_types=[
        pltpu.VMEM((_TP, _D), jnp.float32),
        pltpu.VMEM((_TP, _D), jnp.float32),
        pltpu.VMEM((_BPW, _D), jnp.float32),
        pltpu.VMEM((2, 2, _TP, _D), jnp.float32),
        pltpu.SemaphoreType.DMA((2, 2)),
    ],
)
def _sc_tokenize(x_hbm, w_hbm, b_hbm, o_hbm, w_v, b_v, x_v, o_v, sems):
    wid = lax.axis_index("s") * _NC + lax.axis_index("c")
    base = wid * _BPW
    pltpu.sync_copy(w_hbm, w_v)
    pltpu.sync_copy(b_hbm, b_v)
    pltpu.sync_copy(x_hbm.at[pl.ds(base, _BPW)], x_v)

    def do_pair(p, carry):
        buf = lax.rem(p, 2)

        @pl.when(p >= 2)
        def _wait_prev():
            for k in range(2):
                pltpu.make_async_copy(
                    o_v.at[buf, k, pl.ds(0, _T)],
                    o_hbm.at[base + (p - 2) * 2 + k],
                    sems.at[buf, k],
                ).wait()

        def do_chunk(tc, c2):
            t0 = tc * 16
            xrow0 = x_v[p * 2, pl.ds(t0, 16)]
            xrow1 = x_v[p * 2 + 1, pl.ds(t0, 16)]
            for j in range(16):
                t = t0 + j
                for c in range(8):
                    sl = pl.ds(c * 16, 16)
                    w = w_v[t, sl]
                    bb = b_v[t, sl]
                    o_v[buf, 0, t, sl] = xrow0[j] * w + bb
                    o_v[buf, 1, t, sl] = xrow1[j] * w + bb
            return c2

        lax.fori_loop(0, _TP // 16, do_chunk, 0)

        for k in range(2):
            pltpu.make_async_copy(
                o_v.at[buf, k, pl.ds(0, _T)],
                o_hbm.at[base + p * 2 + k],
                sems.at[buf, k],
            ).start()
        return carry

    lax.fori_loop(0, _PAIRS, do_pair, 0)

    for j in range(2):
        p = _PAIRS - 2 + j
        for k in range(2):
            pltpu.make_async_copy(
                o_v.at[p % 2, k, pl.ds(0, _T)],
                o_hbm.at[base + p * 2 + k],
                sems.at[p % 2, k],
            ).wait()


def kernel(x, weight, bias, cls_token):
    ones = jnp.ones((_B, 1), jnp.float32)
    zcols = jnp.zeros((_B, _D - _T), jnp.float32)
    xpad = jnp.concatenate([ones, x, zcols], axis=1)
    zrows = jnp.zeros((_TP - _T, _D), jnp.float32)
    wpad = jnp.concatenate([cls_token.reshape(1, _D), weight, zrows], axis=0)
    bpad = jnp.concatenate([jnp.zeros((1, _D), jnp.float32), bias, zrows], axis=0)
    return _sc_tokenize(xpad, wpad, bpad)
